# baseline (speedup 1.0000x reference)
import functools
import jax
import jax.numpy as jnp
from jax import lax
from jax.experimental import pallas as pl
from jax.experimental.pallas import tpu as pltpu
from jax.experimental.pallas import tpu_sc as plsc

_H = 16
_EPS = 1e-16
_NC = 2
_NS = 16
_NW = _NC * _NS
_C1 = 400
_C2 = 400
_NINF = float("-inf")


def _mesh():
    return plsc.VectorSubcoreMesh(
        core_axis_name="c", subcore_axis_name="s",
        num_cores=_NC, num_subcores=_NS)


_SC_PARAMS = pltpu.CompilerParams(
    use_tc_tiling_on_sc=False, needs_layout_passes=False)


def _take(x, idx):
    return x.at[idx].get(mode="promise_in_bounds")


@functools.cache
def _p1(e, ns, nd):
    epw = e // _NW
    nchunks = epw // _C1
    ngr = _C1 // 16

    def body(q_hbm, k_hbm, we_hbm, ia_hbm, ib_hbm, ea_hbm,
             alpha_hbm, part_hbm,
             ia0, ia1, ib0, ib1, ea0, ea1, qr0, qr1, kr0, kr1,
             al_v, tab_v, we_v, sl0, sl1, sg0, sg1):
        bufs = ((ia0, ib0, ea0, qr0, kr0, sl0, sg0),
                (ia1, ib1, ea1, qr1, kr1, sl1, sg1))
        c = lax.axis_index("c")
        s = lax.axis_index("s")
        wid = c * _NS + s
        base = wid * epw
        pltpu.sync_copy(we_hbm, we_v)
        wev = we_v[...]
        iot = lax.broadcasted_iota(jnp.int32, (16,), 0)

        def zi(i, cr):
            tab_v[pl.ds(i * 16, 16)] = jnp.full((16,), _NINF, jnp.float32)
            return cr
        lax.fori_loop(0, nd // 16, zi, 0)

        def lin_issue(j, b):
            off = base + j * _C1
            ia, ib, ea = bufs[b][0], bufs[b][1], bufs[b][2]
            sl = bufs[b][5]
            pltpu.async_copy(ia_hbm.at[pl.ds(off, _C1)], ia, sl)
            pltpu.async_copy(ib_hbm.at[pl.ds(off, _C1)], ib, sl)
            pltpu.async_copy(ea_hbm.at[pl.ds(off, _C1)], ea, sl)

        def lin_wait(j, b):
            off = base + j * _C1
            ia, ib, ea = bufs[b][0], bufs[b][1], bufs[b][2]
            sl = bufs[b][5]
            pltpu.make_async_copy(ia_hbm.at[pl.ds(off, _C1)], ia, sl).wait()
            pltpu.make_async_copy(ib_hbm.at[pl.ds(off, _C1)], ib, sl).wait()
            pltpu.make_async_copy(ea_hbm.at[pl.ds(off, _C1)], ea, sl).wait()

        def gat_issue(b):
            ia, ib, qr, kr, sg = (bufs[b][0], bufs[b][1], bufs[b][3],
                                  bufs[b][4], bufs[b][6])
            pltpu.async_copy(q_hbm.at[ia], qr, sg)
            pltpu.async_copy(k_hbm.at[ib], kr, sg)

        def gat_wait(b):
            ia, ib, qr, kr, sg = (bufs[b][0], bufs[b][1], bufs[b][3],
                                  bufs[b][4], bufs[b][6])
            pltpu.make_async_copy(q_hbm.at[ia], qr, sg).wait()
            pltpu.make_async_copy(k_hbm.at[ib], kr, sg).wait()

        def compute(j, b):
            ia, ib, ea, qr, kr = (bufs[b][0], bufs[b][1], bufs[b][2],
                                  bufs[b][3], bufs[b][4])

            def group(g, cr):
                rows = g * 16 + iot
                eag = plsc.load_gather(ea, [rows])
                acc = jnp.zeros((16,), jnp.float32)
                for h in range(_H):
                    col = jnp.full((16,), h, jnp.int32)
                    qc = plsc.load_gather(qr, [rows, col])
                    kc = plsc.load_gather(kr, [rows, col])
                    acc = acc + qc * (kc + wev[h] * eag)
                plsc.store_scatter(al_v, [rows], acc)
                keys = plsc.load_gather(ia, [rows])
                ks, vs = plsc.sort_key_val(keys, acc)
                for sh in (1, 2, 4, 8):
                    up = jnp.maximum(iot - sh, 0)
                    dn = jnp.minimum(iot + sh, 15)
                    vu = jnp.where(_take(ks, up) == ks, _take(vs, up), _NINF)
                    vd = jnp.where(_take(ks, dn) == ks, _take(vs, dn), _NINF)
                    vs = jnp.maximum(vs, jnp.maximum(vu, vd))
                cur = plsc.load_gather(tab_v, [ks])
                plsc.store_scatter(tab_v, [ks], jnp.maximum(cur, vs))
                return cr
            lax.fori_loop(0, ngr, group, 0)
            pltpu.sync_copy(al_v, alpha_hbm.at[pl.ds(base + j * _C1, _C1)])

        lin_issue(0, 0)
        lin_wait(0, 0)
        gat_issue(0)
        lin_issue(1, 1)

        def pair(t, cr):
            j = t * 2
            lin_wait(j + 1, 1)
            gat_issue(1)
            gat_wait(0)
            compute(j, 0)
            lin_issue(j + 2, 0)
            gat_wait(1)
            compute(j + 1, 1)
            lin_wait(j + 2, 0)
            gat_issue(0)
            lin_issue(j + 3, 1)
            return cr
        lax.fori_loop(0, nchunks // 2 - 1, pair, 0)
        j = nchunks - 2
        lin_wait(j + 1, 1)
        gat_issue(1)
        gat_wait(0)
        compute(j, 0)
        gat_wait(1)
        compute(j + 1, 1)
        pltpu.sync_copy(tab_v, part_hbm.at[wid])

    return pl.kernel(
        body,
        out_type=[jax.ShapeDtypeStruct((e,), jnp.float32),
                  jax.ShapeDtypeStruct((_NW, nd), jnp.float32)],
        mesh=_mesh(),
        compiler_params=_SC_PARAMS,
        scratch_types=[
            pltpu.VMEM((_C1,), jnp.int32),
            pltpu.VMEM((_C1,), jnp.int32),
            pltpu.VMEM((_C1,), jnp.int32),
            pltpu.VMEM((_C1,), jnp.int32),
            pltpu.VMEM((_C1,), jnp.float32),
            pltpu.VMEM((_C1,), jnp.float32),
            pltpu.VMEM((_C1, _H), jnp.float32),
            pltpu.VMEM((_C1, _H), jnp.float32),
            pltpu.VMEM((_C1, _H), jnp.float32),
            pltpu.VMEM((_C1, _H), jnp.float32),
            pltpu.VMEM((_C1,), jnp.float32),
            pltpu.VMEM((nd,), jnp.float32),
            pltpu.VMEM((_H,), jnp.float32),
            pltpu.SemaphoreType.DMA,
            pltpu.SemaphoreType.DMA,
            pltpu.SemaphoreType.DMA,
            pltpu.SemaphoreType.DMA,
        ],
    )


@functools.cache
def _p2(e, ns, nd):
    epw = e // _NW
    nchunks = epw // _C2
    ngr = _C2 // 16
    nrow = nd // _NS
    dlen = ((nrow + 7) // 8) * 8
    dnp = dlen * _NS

    def body(v_hbm, am_hbm, ia_hbm, ib_hbm, ea_hbm, al_hbm,
             num_hbm, den_hbm, nea_hbm,
             ia_v, ib_v, ea_v, al_v, msg_v, den_v, nea_v, amr_v,
             am_sh, num_sh, den_sh, nea_sh, sem_v, sem_a):
        c = lax.axis_index("c")
        s = lax.axis_index("s")
        wid = c * _NS + s
        base = wid * epw
        iot = lax.broadcasted_iota(jnp.int32, (16,), 0)
        zero16 = jnp.zeros((16,), jnp.float32)

        def z2(i, cr):
            msg_v[i] = zero16
            return cr
        lax.fori_loop(0, _C2, z2, 0)

        def z1(i, cr):
            den_v[pl.ds(i * 16, 16)] = zero16
            return cr
        lax.fori_loop(0, _C2 // 16, z1, 0)

        d0 = s * dlen
        nfull = dlen // _C2
        drem = dlen - nfull * _C2

        def zs(i, cr):
            pltpu.sync_copy(den_v, den_sh.at[pl.ds(d0 + i * _C2, _C2)])
            pltpu.sync_copy(den_v, nea_sh.at[pl.ds(d0 + i * _C2, _C2)])
            return cr
        lax.fori_loop(0, nfull, zs, 0)
        if drem:
            pltpu.sync_copy(den_v.at[pl.ds(0, drem)],
                            den_sh.at[pl.ds(d0 + nfull * _C2, drem)])
            pltpu.sync_copy(den_v.at[pl.ds(0, drem)],
                            nea_sh.at[pl.ds(d0 + nfull * _C2, drem)])

        r0 = s * nrow

        def z3(i, cr):
            pltpu.sync_copy(msg_v, num_sh.at[pl.ds(r0 + i * _C2, _C2)])
            return cr
        lax.fori_loop(0, nrow // _C2, z3, 0)
        rem = nrow % _C2
        if rem:
            pltpu.sync_copy(msg_v.at[pl.ds(0, rem)],
                            num_sh.at[pl.ds(r0 + (nrow // _C2) * _C2, rem)])

        def za(i, cr):
            pltpu.sync_copy(am_hbm.at[pl.ds(d0 + i * _C2, _C2)], nea_v)
            pltpu.sync_copy(nea_v, am_sh.at[pl.ds(d0 + i * _C2, _C2)])
            return cr
        lax.fori_loop(0, nfull, za, 0)
        if drem:
            pltpu.sync_copy(am_hbm.at[pl.ds(d0 + nfull * _C2, drem)],
                            nea_v.at[pl.ds(0, drem)])
            pltpu.sync_copy(nea_v.at[pl.ds(0, drem)],
                            am_sh.at[pl.ds(d0 + nfull * _C2, drem)])
        plsc.subcore_barrier()

        def group(g, cr):
            rows = g * 16 + iot
            alg = plsc.load_gather(al_v, [rows])
            am = plsc.load_gather(amr_v, [rows])
            ex = jnp.exp(alg - am)
            eag = plsc.load_gather(ea_v, [rows])
            plsc.store_scatter(den_v, [rows], ex)
            plsc.store_scatter(nea_v, [rows], ex * eag)
            for h in range(_H):
                col = jnp.full((16,), h, jnp.int32)
                vc = plsc.load_gather(msg_v, [rows, col])
                plsc.store_scatter(msg_v, [rows, col], vc * ex)
            return cr

        def chunk(j, cr):
            off = base + j * _C2
            pltpu.sync_copy(ia_hbm.at[pl.ds(off, _C2)], ia_v)
            pltpu.sync_copy(al_hbm.at[pl.ds(off, _C2)], al_v)
            pltpu.sync_copy(ib_hbm.at[pl.ds(off, _C2)], ib_v)
            pltpu.sync_copy(ea_hbm.at[pl.ds(off, _C2)], ea_v)
            ca = pltpu.async_copy(am_sh.at[ia_v], amr_v, sem_a)
            cv = pltpu.async_copy(v_hbm.at[ib_v], msg_v, sem_v)
            ca.wait()
            cv.wait()
            lax.fori_loop(0, ngr, group, 0)
            pltpu.sync_copy(msg_v, num_sh.at[ia_v], add=True)
            pltpu.sync_copy(den_v, den_sh.at[ia_v], add=True)
            pltpu.sync_copy(nea_v, nea_sh.at[ia_v], add=True)
            return cr
        lax.fori_loop(0, nchunks, chunk, 0)
        plsc.subcore_barrier()
        pltpu.sync_copy(num_sh.at[pl.ds(r0, nrow)],
                        num_hbm.at[c, pl.ds(r0, nrow)])
        pltpu.sync_copy(den_sh.at[pl.ds(d0, dlen)],
                        den_hbm.at[c, pl.ds(d0, dlen)])
        pltpu.sync_copy(nea_sh.at[pl.ds(d0, dlen)],
                        nea_hbm.at[c, pl.ds(d0, dlen)])

    return pl.kernel(
        body,
        out_type=[jax.ShapeDtypeStruct((_NC, nd, _H), jnp.float32),
                  jax.ShapeDtypeStruct((_NC, dnp), jnp.float32),
                  jax.ShapeDtypeStruct((_NC, dnp), jnp.float32)],
        mesh=_mesh(),
        compiler_params=_SC_PARAMS,
        scratch_types=(
            [pltpu.VMEM((_C2,), jnp.int32)] * 2
            + [pltpu.VMEM((_C2,), jnp.float32)] * 2
            + [pltpu.VMEM((_C2, _H), jnp.float32)]
            + [pltpu.VMEM((_C2,), jnp.float32)] * 3
            + [pltpu.VMEM_SHARED((dnp,), jnp.float32),
               pltpu.VMEM_SHARED((nd, _H), jnp.float32),
               pltpu.VMEM_SHARED((dnp,), jnp.float32),
               pltpu.VMEM_SHARED((dnp,), jnp.float32)]
            + [pltpu.SemaphoreType.DMA] * 2
        ),
    )


def _epilogue_body(num_ref, den_ref, nea_ref, skip_ref, we_ref, out_ref):
    num = num_ref[...]
    den = den_ref[...]
    nea = nea_ref[...]
    out = (num + nea * we_ref[...]) / (den + _EPS) + skip_ref[...]
    out_ref[...] = jnp.maximum(out, 0.0)


def _epilogue(num, den, nea, skip, we):
    n = num.shape[0]
    blk = 2000
    return pl.pallas_call(
        _epilogue_body,
        grid=(n // blk,),
        in_specs=[
            pl.BlockSpec((blk, _H), lambda i: (i, 0)),
            pl.BlockSpec((blk, 1), lambda i: (i, 0)),
            pl.BlockSpec((blk, 1), lambda i: (i, 0)),
            pl.BlockSpec((blk, _H), lambda i: (i, 0)),
            pl.BlockSpec((1, _H), lambda i: (0, 0)),
        ],
        out_specs=pl.BlockSpec((blk, _H), lambda i: (i, 0)),
        out_shape=jax.ShapeDtypeStruct((n, _H), jnp.float32),
    )(num, den[:, None], nea[:, None], skip, we)


def _conv_sc(x_src, x_dst, ia, ib, ea, p):
    ns, nd, e = x_src.shape[0], x_dst.shape[0], ia.shape[0]
    q4 = (x_dst @ p['Wq'] + p['bq']) * 0.25
    k = x_src @ p['Wk'] + p['bk']
    v = x_src @ p['Wv'] + p['bv']
    we = p['We'][0]
    alpha, part = _p1(e, ns, nd)(q4, k, we, ia, ib, ea)
    amax = jnp.max(part, axis=0)
    amax = jnp.where(jnp.isfinite(amax), amax, 0.0)
    dlen = ((nd // _NS + 7) // 8) * 8
    dnp = dlen * _NS
    amax_p = jnp.pad(amax, (0, dnp - nd))
    num2, den2, nea2 = _p2(e, ns, nd)(v, amax_p, ia, ib, ea, alpha)
    num = num2[0] + num2[1]
    den = (den2[0] + den2[1])[:nd]
    nea = (nea2[0] + nea2[1])[:nd]
    skip = x_dst @ p['Wskip'] + p['bskip']
    return _epilogue(num, den, nea, skip, p['We'])


def kernel(x1, x2, edge_index, edge_attr, params):
    src = edge_index[0]
    dst = edge_index[1]
    ea = edge_attr[:, 0]
    h1 = _conv_sc(x2, x1, src, dst, ea, params['c1_w2s'])
    h2 = _conv_sc(x1, x2, dst, src, ea, params['c1_s2w'])
    g1 = _conv_sc(h2, h1, src, dst, ea, params['c2_w2s'])
    g2 = _conv_sc(h1, h2, dst, src, ea, params['c2_s2w'])
    f1 = _conv_sc(g2, g1, src, dst, ea, params['c3_w2s'])
    out = (f1 @ params['fc_W'] + params['fc_b']).squeeze(-1)
    return out


# speedup vs baseline: 1.1317x; 1.1317x over previous
import functools
import jax
import jax.numpy as jnp
from jax import lax
from jax.experimental import pallas as pl
from jax.experimental.pallas import tpu as pltpu
from jax.experimental.pallas import tpu_sc as plsc

_H = 16
_EPS = 1e-16
_NC = 2
_NS = 16
_NW = _NC * _NS
_C1 = 400
_C2 = 400
_NINF = float("-inf")


def _mesh():
    return plsc.VectorSubcoreMesh(
        core_axis_name="c", subcore_axis_name="s",
        num_cores=_NC, num_subcores=_NS)


_SC_PARAMS = pltpu.CompilerParams(
    use_tc_tiling_on_sc=False, needs_layout_passes=False)


def _take(x, idx):
    return x.at[idx].get(mode="promise_in_bounds")


@functools.cache
def _p1(e, ns, nd):
    epw = e // _NW
    nchunks = epw // _C1
    ngr = _C1 // 16

    def body(q_hbm, k_hbm, we_hbm, ia_hbm, ib_hbm, ea_hbm,
             alpha_hbm, part_hbm,
             ia0, ia1, ib0, ib1, ea0, ea1, qr0, qr1, kr0, kr1,
             al_v, tab_v, we_v, sl0, sl1, sg0, sg1):
        bufs = ((ia0, ib0, ea0, qr0, kr0, sl0, sg0),
                (ia1, ib1, ea1, qr1, kr1, sl1, sg1))
        c = lax.axis_index("c")
        s = lax.axis_index("s")
        wid = c * _NS + s
        base = wid * epw
        pltpu.sync_copy(we_hbm, we_v)
        wev = we_v[...]
        iot = lax.broadcasted_iota(jnp.int32, (16,), 0)

        def zi(i, cr):
            tab_v[pl.ds(i * 16, 16)] = jnp.full((16,), _NINF, jnp.float32)
            return cr
        lax.fori_loop(0, nd // 16, zi, 0)

        def lin_issue(j, b):
            off = base + j * _C1
            ia, ib, ea = bufs[b][0], bufs[b][1], bufs[b][2]
            sl = bufs[b][5]
            pltpu.async_copy(ia_hbm.at[pl.ds(off, _C1)], ia, sl)
            pltpu.async_copy(ib_hbm.at[pl.ds(off, _C1)], ib, sl)
            pltpu.async_copy(ea_hbm.at[pl.ds(off, _C1)], ea, sl)

        def lin_wait(j, b):
            off = base + j * _C1
            ia, ib, ea = bufs[b][0], bufs[b][1], bufs[b][2]
            sl = bufs[b][5]
            pltpu.make_async_copy(ia_hbm.at[pl.ds(off, _C1)], ia, sl).wait()
            pltpu.make_async_copy(ib_hbm.at[pl.ds(off, _C1)], ib, sl).wait()
            pltpu.make_async_copy(ea_hbm.at[pl.ds(off, _C1)], ea, sl).wait()

        def gat_issue(b):
            ia, ib, qr, kr, sg = (bufs[b][0], bufs[b][1], bufs[b][3],
                                  bufs[b][4], bufs[b][6])
            pltpu.async_copy(q_hbm.at[ia], qr, sg)
            pltpu.async_copy(k_hbm.at[ib], kr, sg)

        def gat_wait(b):
            ia, ib, qr, kr, sg = (bufs[b][0], bufs[b][1], bufs[b][3],
                                  bufs[b][4], bufs[b][6])
            pltpu.make_async_copy(q_hbm.at[ia], qr, sg).wait()
            pltpu.make_async_copy(k_hbm.at[ib], kr, sg).wait()

        def compute(j, b):
            ia, ib, ea, qr, kr = (bufs[b][0], bufs[b][1], bufs[b][2],
                                  bufs[b][3], bufs[b][4])

            def group(g, cr):
                rows = g * 16 + iot
                eag = plsc.load_gather(ea, [rows])
                acc = jnp.zeros((16,), jnp.float32)
                for h in range(_H):
                    col = jnp.full((16,), h, jnp.int32)
                    qc = plsc.load_gather(qr, [rows, col])
                    kc = plsc.load_gather(kr, [rows, col])
                    acc = acc + qc * (kc + wev[h] * eag)
                plsc.store_scatter(al_v, [rows], acc)
                keys = plsc.load_gather(ia, [rows])
                ks, vs = plsc.sort_key_val(keys, acc)
                for sh in (1, 2, 4, 8):
                    up = jnp.maximum(iot - sh, 0)
                    dn = jnp.minimum(iot + sh, 15)
                    vu = jnp.where(_take(ks, up) == ks, _take(vs, up), _NINF)
                    vd = jnp.where(_take(ks, dn) == ks, _take(vs, dn), _NINF)
                    vs = jnp.maximum(vs, jnp.maximum(vu, vd))
                cur = plsc.load_gather(tab_v, [ks])
                plsc.store_scatter(tab_v, [ks], jnp.maximum(cur, vs))
                return cr
            lax.fori_loop(0, ngr, group, 0)
            pltpu.sync_copy(al_v, alpha_hbm.at[pl.ds(base + j * _C1, _C1)])

        lin_issue(0, 0)
        lin_wait(0, 0)
        gat_issue(0)
        lin_issue(1, 1)

        def pair(t, cr):
            j = t * 2
            lin_wait(j + 1, 1)
            gat_issue(1)
            gat_wait(0)
            compute(j, 0)
            lin_issue(j + 2, 0)
            gat_wait(1)
            compute(j + 1, 1)
            lin_wait(j + 2, 0)
            gat_issue(0)
            lin_issue(j + 3, 1)
            return cr
        lax.fori_loop(0, nchunks // 2 - 1, pair, 0)
        j = nchunks - 2
        lin_wait(j + 1, 1)
        gat_issue(1)
        gat_wait(0)
        compute(j, 0)
        gat_wait(1)
        compute(j + 1, 1)
        pltpu.sync_copy(tab_v, part_hbm.at[wid])

    return pl.kernel(
        body,
        out_type=[jax.ShapeDtypeStruct((e,), jnp.float32),
                  jax.ShapeDtypeStruct((_NW, nd), jnp.float32)],
        mesh=_mesh(),
        compiler_params=_SC_PARAMS,
        scratch_types=[
            pltpu.VMEM((_C1,), jnp.int32),
            pltpu.VMEM((_C1,), jnp.int32),
            pltpu.VMEM((_C1,), jnp.int32),
            pltpu.VMEM((_C1,), jnp.int32),
            pltpu.VMEM((_C1,), jnp.float32),
            pltpu.VMEM((_C1,), jnp.float32),
            pltpu.VMEM((_C1, _H), jnp.float32),
            pltpu.VMEM((_C1, _H), jnp.float32),
            pltpu.VMEM((_C1, _H), jnp.float32),
            pltpu.VMEM((_C1, _H), jnp.float32),
            pltpu.VMEM((_C1,), jnp.float32),
            pltpu.VMEM((nd,), jnp.float32),
            pltpu.VMEM((_H,), jnp.float32),
            pltpu.SemaphoreType.DMA,
            pltpu.SemaphoreType.DMA,
            pltpu.SemaphoreType.DMA,
            pltpu.SemaphoreType.DMA,
        ],
    )


@functools.cache
def _p2(e, ns, nd):
    epw = e // _NW
    nchunks = epw // _C2
    ngr = _C2 // 16
    nrow = nd // _NS
    dlen = ((nrow + 7) // 8) * 8
    dnp = dlen * _NS

    def body(v_hbm, am_hbm, ia_hbm, ib_hbm, ea_hbm, al_hbm,
             num_hbm, den_hbm, nea_hbm,
             ia_v, ib_v, ea_v, al_v, msg_v, den_v, nea_v, amr_v,
             am_sh, num_sh, den_sh, nea_sh, sem_v, sem_a):
        c = lax.axis_index("c")
        s = lax.axis_index("s")
        wid = c * _NS + s
        base = wid * epw
        iot = lax.broadcasted_iota(jnp.int32, (16,), 0)
        zero16 = jnp.zeros((16,), jnp.float32)

        def z2(i, cr):
            msg_v[i] = zero16
            return cr
        lax.fori_loop(0, _C2, z2, 0)

        def z1(i, cr):
            den_v[pl.ds(i * 16, 16)] = zero16
            return cr
        lax.fori_loop(0, _C2 // 16, z1, 0)

        d0 = s * dlen
        nfull = dlen // _C2
        drem = dlen - nfull * _C2

        def zs(i, cr):
            pltpu.sync_copy(den_v, den_sh.at[pl.ds(d0 + i * _C2, _C2)])
            pltpu.sync_copy(den_v, nea_sh.at[pl.ds(d0 + i * _C2, _C2)])
            return cr
        lax.fori_loop(0, nfull, zs, 0)
        if drem:
            pltpu.sync_copy(den_v.at[pl.ds(0, drem)],
                            den_sh.at[pl.ds(d0 + nfull * _C2, drem)])
            pltpu.sync_copy(den_v.at[pl.ds(0, drem)],
                            nea_sh.at[pl.ds(d0 + nfull * _C2, drem)])

        r0 = s * nrow

        def z3(i, cr):
            pltpu.sync_copy(msg_v, num_sh.at[pl.ds(r0 + i * _C2, _C2)])
            return cr
        lax.fori_loop(0, nrow // _C2, z3, 0)
        rem = nrow % _C2
        if rem:
            pltpu.sync_copy(msg_v.at[pl.ds(0, rem)],
                            num_sh.at[pl.ds(r0 + (nrow // _C2) * _C2, rem)])

        def za(i, cr):
            pltpu.sync_copy(am_hbm.at[pl.ds(d0 + i * _C2, _C2)], nea_v)
            pltpu.sync_copy(nea_v, am_sh.at[pl.ds(d0 + i * _C2, _C2)])
            return cr
        lax.fori_loop(0, nfull, za, 0)
        if drem:
            pltpu.sync_copy(am_hbm.at[pl.ds(d0 + nfull * _C2, drem)],
                            nea_v.at[pl.ds(0, drem)])
            pltpu.sync_copy(nea_v.at[pl.ds(0, drem)],
                            am_sh.at[pl.ds(d0 + nfull * _C2, drem)])
        plsc.subcore_barrier()

        def group(g, cr):
            rows = g * 16 + iot
            alg = plsc.load_gather(al_v, [rows])
            am = plsc.load_gather(amr_v, [rows])
            ex = jnp.exp(alg - am)
            eag = plsc.load_gather(ea_v, [rows])
            plsc.store_scatter(den_v, [rows], ex)
            plsc.store_scatter(nea_v, [rows], ex * eag)
            for h in range(_H):
                col = jnp.full((16,), h, jnp.int32)
                vc = plsc.load_gather(msg_v, [rows, col])
                plsc.store_scatter(msg_v, [rows, col], vc * ex)
            return cr

        def chunk(j, cr):
            off = base + j * _C2
            l1 = pltpu.async_copy(ia_hbm.at[pl.ds(off, _C2)], ia_v, sem_v)
            l2 = pltpu.async_copy(al_hbm.at[pl.ds(off, _C2)], al_v, sem_v)
            l3 = pltpu.async_copy(ib_hbm.at[pl.ds(off, _C2)], ib_v, sem_v)
            l4 = pltpu.async_copy(ea_hbm.at[pl.ds(off, _C2)], ea_v, sem_v)
            l1.wait()
            l2.wait()
            l3.wait()
            l4.wait()
            ca = pltpu.async_copy(am_sh.at[ia_v], amr_v, sem_a)
            cv = pltpu.async_copy(v_hbm.at[ib_v], msg_v, sem_v)
            ca.wait()
            cv.wait()
            lax.fori_loop(0, ngr, group, 0)
            pltpu.sync_copy(msg_v, num_sh.at[ia_v], add=True)
            pltpu.sync_copy(den_v, den_sh.at[ia_v], add=True)
            pltpu.sync_copy(nea_v, nea_sh.at[ia_v], add=True)
            return cr
        lax.fori_loop(0, nchunks, chunk, 0)
        plsc.subcore_barrier()
        pltpu.sync_copy(num_sh.at[pl.ds(r0, nrow)],
                        num_hbm.at[c, pl.ds(r0, nrow)])
        pltpu.sync_copy(den_sh.at[pl.ds(d0, dlen)],
                        den_hbm.at[c, pl.ds(d0, dlen)])
        pltpu.sync_copy(nea_sh.at[pl.ds(d0, dlen)],
                        nea_hbm.at[c, pl.ds(d0, dlen)])

    return pl.kernel(
        body,
        out_type=[jax.ShapeDtypeStruct((_NC, nd, _H), jnp.float32),
                  jax.ShapeDtypeStruct((_NC, dnp), jnp.float32),
                  jax.ShapeDtypeStruct((_NC, dnp), jnp.float32)],
        mesh=_mesh(),
        compiler_params=_SC_PARAMS,
        scratch_types=(
            [pltpu.VMEM((_C2,), jnp.int32)] * 2
            + [pltpu.VMEM((_C2,), jnp.float32)] * 2
            + [pltpu.VMEM((_C2, _H), jnp.float32)]
            + [pltpu.VMEM((_C2,), jnp.float32)] * 3
            + [pltpu.VMEM_SHARED((dnp,), jnp.float32),
               pltpu.VMEM_SHARED((nd, _H), jnp.float32),
               pltpu.VMEM_SHARED((dnp,), jnp.float32),
               pltpu.VMEM_SHARED((dnp,), jnp.float32)]
            + [pltpu.SemaphoreType.DMA] * 2
        ),
    )


def _epilogue_body(num_ref, den_ref, nea_ref, skip_ref, we_ref, out_ref):
    num = num_ref[...]
    den = den_ref[...]
    nea = nea_ref[...]
    out = (num + nea * we_ref[...]) / (den + _EPS) + skip_ref[...]
    out_ref[...] = jnp.maximum(out, 0.0)


def _epilogue(num, den, nea, skip, we):
    n = num.shape[0]
    blk = 2000
    return pl.pallas_call(
        _epilogue_body,
        grid=(n // blk,),
        in_specs=[
            pl.BlockSpec((blk, _H), lambda i: (i, 0)),
            pl.BlockSpec((blk, 1), lambda i: (i, 0)),
            pl.BlockSpec((blk, 1), lambda i: (i, 0)),
            pl.BlockSpec((blk, _H), lambda i: (i, 0)),
            pl.BlockSpec((1, _H), lambda i: (0, 0)),
        ],
        out_specs=pl.BlockSpec((blk, _H), lambda i: (i, 0)),
        out_shape=jax.ShapeDtypeStruct((n, _H), jnp.float32),
    )(num, den[:, None], nea[:, None], skip, we)


def _conv_sc(x_src, x_dst, ia, ib, ea, p):
    ns, nd, e = x_src.shape[0], x_dst.shape[0], ia.shape[0]
    q4 = (x_dst @ p['Wq'] + p['bq']) * 0.25
    k = x_src @ p['Wk'] + p['bk']
    v = x_src @ p['Wv'] + p['bv']
    we = p['We'][0]
    alpha, part = _p1(e, ns, nd)(q4, k, we, ia, ib, ea)
    amax = jnp.max(part, axis=0)
    amax = jnp.where(jnp.isfinite(amax), amax, 0.0)
    dlen = ((nd // _NS + 7) // 8) * 8
    dnp = dlen * _NS
    amax_p = jnp.pad(amax, (0, dnp - nd))
    num2, den2, nea2 = _p2(e, ns, nd)(v, amax_p, ia, ib, ea, alpha)
    num = num2[0] + num2[1]
    den = (den2[0] + den2[1])[:nd]
    nea = (nea2[0] + nea2[1])[:nd]
    skip = x_dst @ p['Wskip'] + p['bskip']
    return _epilogue(num, den, nea, skip, p['We'])


def kernel(x1, x2, edge_index, edge_attr, params):
    src = edge_index[0]
    dst = edge_index[1]
    ea = edge_attr[:, 0]
    h1 = _conv_sc(x2, x1, src, dst, ea, params['c1_w2s'])
    h2 = _conv_sc(x1, x2, dst, src, ea, params['c1_s2w'])
    g1 = _conv_sc(h2, h1, src, dst, ea, params['c2_w2s'])
    g2 = _conv_sc(h1, h2, dst, src, ea, params['c2_s2w'])
    f1 = _conv_sc(g2, g1, src, dst, ea, params['c3_w2s'])
    out = (f1 @ params['fc_W'] + params['fc_b']).squeeze(-1)
    return out


# speedup vs baseline: 1.1453x; 1.0120x over previous
import functools
import jax
import jax.numpy as jnp
from jax import lax
from jax.experimental import pallas as pl
from jax.experimental.pallas import tpu as pltpu
from jax.experimental.pallas import tpu_sc as plsc

_H = 16
_EPS = 1e-16
_NC = 2
_NS = 16
_NW = _NC * _NS
_C1 = 400
_C2 = 400
_NINF = float("-inf")


def _mesh():
    return plsc.VectorSubcoreMesh(
        core_axis_name="c", subcore_axis_name="s",
        num_cores=_NC, num_subcores=_NS)


_SC_PARAMS = pltpu.CompilerParams(
    use_tc_tiling_on_sc=False, needs_layout_passes=False)


def _take(x, idx):
    return x.at[idx].get(mode="promise_in_bounds")


@functools.cache
def _p1(e, ns, nd):
    epw = e // _NW
    nchunks = epw // _C1
    ngr = _C1 // 16

    def body(q_hbm, k_hbm, we_hbm, ia_hbm, ib_hbm, ea_hbm,
             alpha_hbm, part_hbm,
             ia0, ia1, ib0, ib1, ea0, ea1, qr0, qr1, kr0, kr1,
             al_v, tab_v, we_v, sl0, sl1, sg0, sg1):
        bufs = ((ia0, ib0, ea0, qr0, kr0, sl0, sg0),
                (ia1, ib1, ea1, qr1, kr1, sl1, sg1))
        c = lax.axis_index("c")
        s = lax.axis_index("s")
        wid = c * _NS + s
        base = wid * epw
        pltpu.sync_copy(we_hbm, we_v)
        wev = we_v[...]
        iot = lax.broadcasted_iota(jnp.int32, (16,), 0)

        def zi(i, cr):
            tab_v[pl.ds(i * 16, 16)] = jnp.full((16,), _NINF, jnp.float32)
            return cr
        lax.fori_loop(0, nd // 16, zi, 0)

        def lin_issue(j, b):
            off = base + j * _C1
            ia, ib, ea = bufs[b][0], bufs[b][1], bufs[b][2]
            sl = bufs[b][5]
            pltpu.async_copy(ia_hbm.at[pl.ds(off, _C1)], ia, sl)
            pltpu.async_copy(ib_hbm.at[pl.ds(off, _C1)], ib, sl)
            pltpu.async_copy(ea_hbm.at[pl.ds(off, _C1)], ea, sl)

        def lin_wait(j, b):
            off = base + j * _C1
            ia, ib, ea = bufs[b][0], bufs[b][1], bufs[b][2]
            sl = bufs[b][5]
            pltpu.make_async_copy(ia_hbm.at[pl.ds(off, _C1)], ia, sl).wait()
            pltpu.make_async_copy(ib_hbm.at[pl.ds(off, _C1)], ib, sl).wait()
            pltpu.make_async_copy(ea_hbm.at[pl.ds(off, _C1)], ea, sl).wait()

        def gat_issue(b):
            ia, ib, qr, kr, sg = (bufs[b][0], bufs[b][1], bufs[b][3],
                                  bufs[b][4], bufs[b][6])
            pltpu.async_copy(q_hbm.at[ia], qr, sg)
            pltpu.async_copy(k_hbm.at[ib], kr, sg)

        def gat_wait(b):
            ia, ib, qr, kr, sg = (bufs[b][0], bufs[b][1], bufs[b][3],
                                  bufs[b][4], bufs[b][6])
            pltpu.make_async_copy(q_hbm.at[ia], qr, sg).wait()
            pltpu.make_async_copy(k_hbm.at[ib], kr, sg).wait()

        def compute(j, b):
            ia, ib, ea, qr, kr = (bufs[b][0], bufs[b][1], bufs[b][2],
                                  bufs[b][3], bufs[b][4])

            def group(g, cr):
                rows = g * 16 + iot
                eag = plsc.load_gather(ea, [rows])
                acc = jnp.zeros((16,), jnp.float32)
                for h in range(_H):
                    col = jnp.full((16,), h, jnp.int32)
                    qc = plsc.load_gather(qr, [rows, col])
                    kc = plsc.load_gather(kr, [rows, col])
                    acc = acc + qc * (kc + wev[h] * eag)
                plsc.store_scatter(al_v, [rows], acc)
                keys = plsc.load_gather(ia, [rows])
                ks, vs = plsc.sort_key_val(keys, acc)
                for sh in (1, 2, 4, 8):
                    up = jnp.maximum(iot - sh, 0)
                    dn = jnp.minimum(iot + sh, 15)
                    vu = jnp.where(_take(ks, up) == ks, _take(vs, up), _NINF)
                    vd = jnp.where(_take(ks, dn) == ks, _take(vs, dn), _NINF)
                    vs = jnp.maximum(vs, jnp.maximum(vu, vd))
                cur = plsc.load_gather(tab_v, [ks])
                plsc.store_scatter(tab_v, [ks], jnp.maximum(cur, vs))
                return cr
            lax.fori_loop(0, ngr, group, 0)
            pltpu.sync_copy(al_v, alpha_hbm.at[pl.ds(base + j * _C1, _C1)])

        lin_issue(0, 0)
        lin_wait(0, 0)
        gat_issue(0)
        lin_issue(1, 1)

        def pair(t, cr):
            j = t * 2
            lin_wait(j + 1, 1)
            gat_issue(1)
            gat_wait(0)
            compute(j, 0)
            lin_issue(j + 2, 0)
            gat_wait(1)
            compute(j + 1, 1)
            lin_wait(j + 2, 0)
            gat_issue(0)
            lin_issue(j + 3, 1)
            return cr
        lax.fori_loop(0, nchunks // 2 - 1, pair, 0)
        j = nchunks - 2
        lin_wait(j + 1, 1)
        gat_issue(1)
        gat_wait(0)
        compute(j, 0)
        gat_wait(1)
        compute(j + 1, 1)
        pltpu.sync_copy(tab_v, part_hbm.at[wid])

    return pl.kernel(
        body,
        out_type=[jax.ShapeDtypeStruct((e,), jnp.float32),
                  jax.ShapeDtypeStruct((_NW, nd), jnp.float32)],
        mesh=_mesh(),
        compiler_params=_SC_PARAMS,
        scratch_types=[
            pltpu.VMEM((_C1,), jnp.int32),
            pltpu.VMEM((_C1,), jnp.int32),
            pltpu.VMEM((_C1,), jnp.int32),
            pltpu.VMEM((_C1,), jnp.int32),
            pltpu.VMEM((_C1,), jnp.float32),
            pltpu.VMEM((_C1,), jnp.float32),
            pltpu.VMEM((_C1, _H), jnp.float32),
            pltpu.VMEM((_C1, _H), jnp.float32),
            pltpu.VMEM((_C1, _H), jnp.float32),
            pltpu.VMEM((_C1, _H), jnp.float32),
            pltpu.VMEM((_C1,), jnp.float32),
            pltpu.VMEM((nd,), jnp.float32),
            pltpu.VMEM((_H,), jnp.float32),
            pltpu.SemaphoreType.DMA,
            pltpu.SemaphoreType.DMA,
            pltpu.SemaphoreType.DMA,
            pltpu.SemaphoreType.DMA,
        ],
    )


@functools.cache
def _p2(e, ns, nd):
    epw = e // _NW
    nchunks = epw // _C2
    ngr = _C2 // 16
    nrow = nd // _NS
    dlen = ((nrow + 7) // 8) * 8
    dnp = dlen * _NS

    def body(v_hbm, am_hbm, ia_hbm, ib_hbm, ea_hbm, al_hbm,
             num_hbm, den_hbm, nea_hbm,
             ia_v, ib_v, ea_v, al_v, msg_v, den_v, nea_v, amr_v,
             am_sh, num_sh, den_sh, nea_sh, sem_v, sem_a):
        c = lax.axis_index("c")
        s = lax.axis_index("s")
        wid = c * _NS + s
        base = wid * epw
        iot = lax.broadcasted_iota(jnp.int32, (16,), 0)
        zero16 = jnp.zeros((16,), jnp.float32)

        def z2(i, cr):
            msg_v[i] = zero16
            return cr
        lax.fori_loop(0, _C2, z2, 0)

        def z1(i, cr):
            den_v[pl.ds(i * 16, 16)] = zero16
            return cr
        lax.fori_loop(0, _C2 // 16, z1, 0)

        d0 = s * dlen
        nfull = dlen // _C2
        drem = dlen - nfull * _C2

        def zs(i, cr):
            pltpu.sync_copy(den_v, den_sh.at[pl.ds(d0 + i * _C2, _C2)])
            pltpu.sync_copy(den_v, nea_sh.at[pl.ds(d0 + i * _C2, _C2)])
            return cr
        lax.fori_loop(0, nfull, zs, 0)
        if drem:
            pltpu.sync_copy(den_v.at[pl.ds(0, drem)],
                            den_sh.at[pl.ds(d0 + nfull * _C2, drem)])
            pltpu.sync_copy(den_v.at[pl.ds(0, drem)],
                            nea_sh.at[pl.ds(d0 + nfull * _C2, drem)])

        r0 = s * nrow

        def z3(i, cr):
            pltpu.sync_copy(msg_v, num_sh.at[pl.ds(r0 + i * _C2, _C2)])
            return cr
        lax.fori_loop(0, nrow // _C2, z3, 0)
        rem = nrow % _C2
        if rem:
            pltpu.sync_copy(msg_v.at[pl.ds(0, rem)],
                            num_sh.at[pl.ds(r0 + (nrow // _C2) * _C2, rem)])

        def za(i, cr):
            pltpu.sync_copy(am_hbm.at[pl.ds(d0 + i * _C2, _C2)], nea_v)
            pltpu.sync_copy(nea_v, am_sh.at[pl.ds(d0 + i * _C2, _C2)])
            return cr
        lax.fori_loop(0, nfull, za, 0)
        if drem:
            pltpu.sync_copy(am_hbm.at[pl.ds(d0 + nfull * _C2, drem)],
                            nea_v.at[pl.ds(0, drem)])
            pltpu.sync_copy(nea_v.at[pl.ds(0, drem)],
                            am_sh.at[pl.ds(d0 + nfull * _C2, drem)])
        plsc.subcore_barrier()

        def group(g, cr):
            rows = g * 16 + iot
            alg = plsc.load_gather(al_v, [rows])
            am = plsc.load_gather(amr_v, [rows])
            ex = jnp.exp(alg - am)
            eag = plsc.load_gather(ea_v, [rows])
            plsc.store_scatter(den_v, [rows], ex)
            plsc.store_scatter(nea_v, [rows], ex * eag)
            for h in range(_H):
                col = jnp.full((16,), h, jnp.int32)
                vc = plsc.load_gather(msg_v, [rows, col])
                plsc.store_scatter(msg_v, [rows, col], vc * ex)
            return cr

        def chunk(j, cr):
            off = base + j * _C2
            l1 = pltpu.async_copy(ia_hbm.at[pl.ds(off, _C2)], ia_v, sem_v)
            l2 = pltpu.async_copy(al_hbm.at[pl.ds(off, _C2)], al_v, sem_v)
            l3 = pltpu.async_copy(ib_hbm.at[pl.ds(off, _C2)], ib_v, sem_v)
            l4 = pltpu.async_copy(ea_hbm.at[pl.ds(off, _C2)], ea_v, sem_v)
            l1.wait()
            l2.wait()
            l3.wait()
            l4.wait()
            ca = pltpu.async_copy(am_sh.at[ia_v], amr_v, sem_a)
            cv = pltpu.async_copy(v_hbm.at[ib_v], msg_v, sem_v)
            ca.wait()
            cv.wait()
            lax.fori_loop(0, ngr, group, 0)
            s1 = pltpu.async_copy(msg_v, num_sh.at[ia_v], sem_a, add=True)
            s2 = pltpu.async_copy(den_v, den_sh.at[ia_v], sem_a, add=True)
            s3 = pltpu.async_copy(nea_v, nea_sh.at[ia_v], sem_a, add=True)
            s1.wait()
            s2.wait()
            s3.wait()
            return cr
        lax.fori_loop(0, nchunks, chunk, 0)
        plsc.subcore_barrier()
        pltpu.sync_copy(num_sh.at[pl.ds(r0, nrow)],
                        num_hbm.at[c, pl.ds(r0, nrow)])
        pltpu.sync_copy(den_sh.at[pl.ds(d0, dlen)],
                        den_hbm.at[c, pl.ds(d0, dlen)])
        pltpu.sync_copy(nea_sh.at[pl.ds(d0, dlen)],
                        nea_hbm.at[c, pl.ds(d0, dlen)])

    return pl.kernel(
        body,
        out_type=[jax.ShapeDtypeStruct((_NC, nd, _H), jnp.float32),
                  jax.ShapeDtypeStruct((_NC, dnp), jnp.float32),
                  jax.ShapeDtypeStruct((_NC, dnp), jnp.float32)],
        mesh=_mesh(),
        compiler_params=_SC_PARAMS,
        scratch_types=(
            [pltpu.VMEM((_C2,), jnp.int32)] * 2
            + [pltpu.VMEM((_C2,), jnp.float32)] * 2
            + [pltpu.VMEM((_C2, _H), jnp.float32)]
            + [pltpu.VMEM((_C2,), jnp.float32)] * 3
            + [pltpu.VMEM_SHARED((dnp,), jnp.float32),
               pltpu.VMEM_SHARED((nd, _H), jnp.float32),
               pltpu.VMEM_SHARED((dnp,), jnp.float32),
               pltpu.VMEM_SHARED((dnp,), jnp.float32)]
            + [pltpu.SemaphoreType.DMA] * 2
        ),
    )


def _epilogue_body(num_ref, den_ref, nea_ref, skip_ref, we_ref, out_ref):
    num = num_ref[...]
    den = den_ref[...]
    nea = nea_ref[...]
    out = (num + nea * we_ref[...]) / (den + _EPS) + skip_ref[...]
    out_ref[...] = jnp.maximum(out, 0.0)


def _epilogue(num, den, nea, skip, we):
    n = num.shape[0]
    blk = 2000
    return pl.pallas_call(
        _epilogue_body,
        grid=(n // blk,),
        in_specs=[
            pl.BlockSpec((blk, _H), lambda i: (i, 0)),
            pl.BlockSpec((blk, 1), lambda i: (i, 0)),
            pl.BlockSpec((blk, 1), lambda i: (i, 0)),
            pl.BlockSpec((blk, _H), lambda i: (i, 0)),
            pl.BlockSpec((1, _H), lambda i: (0, 0)),
        ],
        out_specs=pl.BlockSpec((blk, _H), lambda i: (i, 0)),
        out_shape=jax.ShapeDtypeStruct((n, _H), jnp.float32),
    )(num, den[:, None], nea[:, None], skip, we)


def _conv_sc(x_src, x_dst, ia, ib, ea, p):
    ns, nd, e = x_src.shape[0], x_dst.shape[0], ia.shape[0]
    q4 = (x_dst @ p['Wq'] + p['bq']) * 0.25
    k = x_src @ p['Wk'] + p['bk']
    v = x_src @ p['Wv'] + p['bv']
    we = p['We'][0]
    alpha, part = _p1(e, ns, nd)(q4, k, we, ia, ib, ea)
    amax = jnp.max(part, axis=0)
    amax = jnp.where(jnp.isfinite(amax), amax, 0.0)
    dlen = ((nd // _NS + 7) // 8) * 8
    dnp = dlen * _NS
    amax_p = jnp.pad(amax, (0, dnp - nd))
    num2, den2, nea2 = _p2(e, ns, nd)(v, amax_p, ia, ib, ea, alpha)
    num = num2[0] + num2[1]
    den = (den2[0] + den2[1])[:nd]
    nea = (nea2[0] + nea2[1])[:nd]
    skip = x_dst @ p['Wskip'] + p['bskip']
    return _epilogue(num, den, nea, skip, p['We'])


def kernel(x1, x2, edge_index, edge_attr, params):
    src = edge_index[0]
    dst = edge_index[1]
    ea = edge_attr[:, 0]
    h1 = _conv_sc(x2, x1, src, dst, ea, params['c1_w2s'])
    h2 = _conv_sc(x1, x2, dst, src, ea, params['c1_s2w'])
    g1 = _conv_sc(h2, h1, src, dst, ea, params['c2_w2s'])
    g2 = _conv_sc(h1, h2, dst, src, ea, params['c2_s2w'])
    f1 = _conv_sc(g2, g1, src, dst, ea, params['c3_w2s'])
    out = (f1 @ params['fc_W'] + params['fc_b']).squeeze(-1)
    return out
